# baseline (speedup 1.0000x reference)
import functools

import jax
import jax.numpy as jnp
from jax import lax
from jax.experimental import pallas as pl
from jax.experimental.pallas import tpu as pltpu
from jax.experimental.pallas import tpu_sc as plsc

_LANES = 16
_GATHER_CHUNK = 128


@jax.jit
def _sc_embed_call(ids, ptab, ttab, p2t):
    B = ids.shape[0]
    V, D = ptab.shape

    info = plsc.get_sparse_core_info()
    NC, NS = info.num_cores, info.num_subcores
    NW = NC * NS
    bpw = B // NW
    n_ch = bpw // _GATHER_CHUNK

    mesh = plsc.VectorSubcoreMesh(core_axis_name="c", subcore_axis_name="s")

    @functools.partial(
        pl.kernel,
        mesh=mesh,
        compiler_params=pltpu.CompilerParams(use_tc_tiling_on_sc=False),
        out_type=jax.ShapeDtypeStruct((B, D), jnp.float32),
        scratch_types=[
            pltpu.VMEM((bpw,), jnp.int32),
            pltpu.VMEM((bpw,), jnp.int32),
            pltpu.VMEM((bpw, D), jnp.float32),
            pltpu.VMEM((bpw, D), jnp.float32),
            pltpu.SemaphoreType.DMA,
            pltpu.SemaphoreType.DMA,
        ],
    )
    def sc_embed(pid_hbm, ptab_hbm, ttab_hbm, p2t_hbm, out_hbm,
                 idx_v, tids_v, rows_v, trows_v, sem_rows, sem_tids):
        wid = lax.axis_index("s") * NC + lax.axis_index("c")
        base = wid * bpw
        pltpu.sync_copy(pid_hbm.at[pl.ds(base, bpw)], idx_v)

        row_copies = []
        tid_copies = []
        for c in range(n_ch):
            sl = pl.ds(c * _GATHER_CHUNK, _GATHER_CHUNK)
            row_copies.append(
                pltpu.async_copy(ptab_hbm.at[idx_v.at[sl]], rows_v.at[sl],
                                 sem_rows))
            tid_copies.append(
                pltpu.async_copy(p2t_hbm.at[idx_v.at[sl]], tids_v.at[sl],
                                 sem_tids))
        for cp in tid_copies:
            cp.wait()
        t_copies = []
        for c in range(n_ch):
            sl = pl.ds(c * _GATHER_CHUNK, _GATHER_CHUNK)
            t_copies.append(
                pltpu.async_copy(ttab_hbm.at[tids_v.at[sl]], trows_v.at[sl],
                                 sem_tids))
        for cp in row_copies:
            cp.wait()
        for cp in t_copies:
            cp.wait()

        @pl.loop(0, bpw)
        def _(i):
            for j in range(0, D, _LANES):
                sl = pl.ds(j, _LANES)
                rows_v[i, sl] += trows_v[i, sl]

        pltpu.sync_copy(rows_v, out_hbm.at[pl.ds(base, bpw)])

    return sc_embed(ids, ptab, ttab, p2t)


def kernel(primitive_ids, primitive_embeddings, type_embeddings,
           primitive_to_type):
    ids = primitive_ids.astype(jnp.int32)
    p2t = primitive_to_type.astype(jnp.int32)
    return _sc_embed_call(ids, primitive_embeddings, type_embeddings, p2t)


# speedup vs baseline: 2.5315x; 2.5315x over previous
import functools

import jax
import jax.numpy as jnp
from jax import lax
from jax.experimental import pallas as pl
from jax.experimental.pallas import tpu as pltpu
from jax.experimental.pallas import tpu_sc as plsc

_LANES = 16
_GATHER_CHUNK = 128


@jax.jit
def _sc_embed_call(ids, ptab, ttab, p2t):
    B = ids.shape[0]
    V, D = ptab.shape
    ttab_shape = ttab.shape

    info = plsc.get_sparse_core_info()
    NC, NS = info.num_cores, info.num_subcores
    NW = NC * NS
    bpw = B // NW
    n_ch = bpw // _GATHER_CHUNK

    mesh = plsc.VectorSubcoreMesh(core_axis_name="c", subcore_axis_name="s")

    @functools.partial(
        pl.kernel,
        mesh=mesh,
        compiler_params=pltpu.CompilerParams(use_tc_tiling_on_sc=False),
        out_type=jax.ShapeDtypeStruct((B, D), jnp.float32),
        scratch_types=[
            pltpu.VMEM((bpw,), jnp.int32),
            pltpu.VMEM((bpw,), jnp.int32),
            pltpu.VMEM((bpw, D), jnp.float32),
            pltpu.VMEM(ttab_shape, jnp.float32),
            pltpu.SemaphoreType.DMA,
            pltpu.SemaphoreType.DMA,
        ],
    )
    def sc_embed(pid_hbm, ptab_hbm, ttab_hbm, p2t_hbm, out_hbm,
                 idx_v, tids_v, rows_v, ttab_v, sem_rows, sem_tids):
        wid = lax.axis_index("s") * NC + lax.axis_index("c")
        base = wid * bpw
        pltpu.sync_copy(pid_hbm.at[pl.ds(base, bpw)], idx_v)

        row_copies = []
        tid_copies = []
        for c in range(n_ch):
            sl = pl.ds(c * _GATHER_CHUNK, _GATHER_CHUNK)
            row_copies.append(
                pltpu.async_copy(ptab_hbm.at[idx_v.at[sl]], rows_v.at[sl],
                                 sem_rows))
            tid_copies.append(
                pltpu.async_copy(p2t_hbm.at[idx_v.at[sl]], tids_v.at[sl],
                                 sem_tids))
        pltpu.sync_copy(ttab_hbm, ttab_v)
        for cp in tid_copies:
            cp.wait()
        for cp in row_copies:
            cp.wait()

        @functools.partial(plsc.parallel_loop, 0, bpw, unroll=8)
        def _(i):
            t = tids_v[i]
            for j in range(0, D, _LANES):
                sl = pl.ds(j, _LANES)
                rows_v[i, sl] += ttab_v[t, sl]

        pltpu.sync_copy(rows_v, out_hbm.at[pl.ds(base, bpw)])

    return sc_embed(ids, ptab, ttab, p2t)


def kernel(primitive_ids, primitive_embeddings, type_embeddings,
           primitive_to_type):
    ids = primitive_ids.astype(jnp.int32)
    p2t = primitive_to_type.astype(jnp.int32)
    return _sc_embed_call(ids, primitive_embeddings, type_embeddings, p2t)
